# Initial kernel scaffold; baseline (speedup 1.0000x reference)
import jax
import jax.numpy as jnp
from jax.experimental import pallas as pl


def kernel(edge_lengths, edge_index, pos, W1_0, b1_0, W2_0, b2_0, W1_1, b1_1, W2_1, b2_1, W1_2, b1_2, W2_2, b2_2):
    raise NotImplementedError("write your pallas kernel here")


# speedup vs baseline: 2.2676x; 2.2676x over previous
import functools

import jax
import jax.numpy as jnp
from jax import lax
from jax.experimental import pallas as pl
from jax.experimental.pallas import tpu as pltpu
from jax.experimental.pallas import tpu_sc as plsc

N_NODES = 10000
N_EDGES = 320000
D = 128
GAMMA = 10.0
C_SCALE = 6.0 / 127.0


EDGE_BLK = 2000


def _edge_feat_body(d_ref, w1_ref, b1_ref, w2_ref, b2_ref, out_ref):
    d = d_ref[...]
    centers = lax.broadcasted_iota(jnp.int32, (1, D), 1).astype(jnp.float32) * C_SCALE
    diff = d - centers
    rbf = jnp.exp((-GAMMA) * diff * diff)

    feats = []
    for k in range(3):
        h = jnp.maximum(
            jnp.dot(rbf, w1_ref[k], preferred_element_type=jnp.float32)
            + b1_ref[k, :][None, :],
            0.0,
        )
        f = (
            jnp.dot(h, w2_ref[k], preferred_element_type=jnp.float32)
            + b2_ref[k, :][None, :]
        )
        feats.append(f)

    m1 = d >= 3.0
    m2 = d >= 4.5
    out_ref[...] = jnp.where(m2, feats[2], jnp.where(m1, feats[1], feats[0]))


def _edge_feat(d_col, w1, b1, w2, b2):
    grid = N_EDGES // EDGE_BLK
    return pl.pallas_call(
        _edge_feat_body,
        grid=(grid,),
        in_specs=[
            pl.BlockSpec((EDGE_BLK, 1), lambda i: (i, 0)),
            pl.BlockSpec((3, D, D), lambda i: (0, 0, 0)),
            pl.BlockSpec((3, D), lambda i: (0, 0)),
            pl.BlockSpec((3, D, D), lambda i: (0, 0, 0)),
            pl.BlockSpec((3, D), lambda i: (0, 0)),
        ],
        out_specs=pl.BlockSpec((EDGE_BLK, D), lambda i: (i, 0)),
        out_shape=jax.ShapeDtypeStruct((N_EDGES, D), jnp.float32),
        compiler_params=pltpu.CompilerParams(
            dimension_semantics=("arbitrary",),
        ),
    )(d_col, w1, b1, w2, b2)


NC, NS = 2, 16
NW = NC * NS
E_PER_W = N_EDGES // NW
CHUNK = 80
N_CHUNKS = E_PER_W // CHUNK
ACC_ROWS = 10240
ZROWS = 16
DUMMY = N_NODES
OUT_ROWS = 624


def _seg_body(len_hbm, dst_hbm, feat_hbm, out_hbm,
              len_v, dst_v, idx_v, feat_v, zero_v, acc_s):
    core = lax.axis_index("c")
    sid = lax.axis_index("s")
    wid = core * NS + sid
    base = wid * E_PER_W

    for r in range(8):
        zero_v[pl.ds(r * 2, 2), :] = jnp.zeros((2, D), jnp.float32)
    zbase = sid * (ACC_ROWS // NS)

    def _zero(j, _):
        pltpu.sync_copy(zero_v, acc_s.at[pl.ds(zbase + j * ZROWS, ZROWS)])
        return 0

    lax.fori_loop(0, (ACC_ROWS // NS) // ZROWS, _zero, 0)

    pltpu.sync_copy(len_hbm.at[pl.ds(base, E_PER_W)], len_v)
    pltpu.sync_copy(dst_hbm.at[pl.ds(base, E_PER_W)], dst_v)

    def _mkidx(j, _):
        for k in range(CHUNK // 16):
            off = j * CHUNK + k * 16
            lv = len_v[pl.ds(off, 16)]
            dv = dst_v[pl.ds(off, 16)]
            idx_v[j, pl.ds(k * 16, 16)] = jnp.where(
                lv < 3.0, dv, jnp.full((16,), DUMMY, jnp.int32)
            )
        return 0

    lax.fori_loop(0, N_CHUNKS, _mkidx, 0)

    plsc.subcore_barrier()

    def _scat(j, _):
        pltpu.sync_copy(feat_hbm.at[pl.ds(base + j * CHUNK, CHUNK)], feat_v)
        pltpu.sync_copy(feat_v, acc_s.at[idx_v.at[j]], add=True)
        return 0

    lax.fori_loop(0, N_CHUNKS, _scat, 0)

    plsc.subcore_barrier()

    obase = sid * OUT_ROWS
    pltpu.sync_copy(
        acc_s.at[pl.ds(obase, OUT_ROWS)],
        out_hbm.at[core, pl.ds(obase, OUT_ROWS)],
    )

    @pl.when(sid == 0)
    def _tail():
        pltpu.sync_copy(
            acc_s.at[pl.ds(NS * OUT_ROWS, N_NODES - NS * OUT_ROWS)],
            out_hbm.at[core, pl.ds(NS * OUT_ROWS, N_NODES - NS * OUT_ROWS)],
        )


@functools.partial(jax.jit, static_argnums=())
def _segment_partials(edge_lengths, dst, edge_feat):
    mesh = plsc.VectorSubcoreMesh(core_axis_name="c", subcore_axis_name="s")
    f = pl.kernel(
        _seg_body,
        out_type=jax.ShapeDtypeStruct((NC, N_NODES, D), jnp.float32),
        mesh=mesh,
        scratch_types=[
            pltpu.VMEM((E_PER_W,), jnp.float32),
            pltpu.VMEM((E_PER_W,), jnp.int32),
            pltpu.VMEM((N_CHUNKS, CHUNK), jnp.int32),
            pltpu.VMEM((CHUNK, D), jnp.float32),
            pltpu.VMEM((ZROWS, D), jnp.float32),
            pltpu.VMEM_SHARED((ACC_ROWS, D), jnp.float32),
        ],
    )
    return f(edge_lengths, dst, edge_feat)


def kernel(edge_lengths, edge_index, pos,
           W1_0, b1_0, W2_0, b2_0,
           W1_1, b1_1, W2_1, b2_1,
           W1_2, b1_2, W2_2, b2_2):
    w1 = jnp.stack([W1_0, W1_1, W1_2])
    b1 = jnp.stack([b1_0, b1_1, b1_2])
    w2 = jnp.stack([W2_0, W2_1, W2_2])
    b2 = jnp.stack([b2_0, b2_1, b2_2])
    d_col = edge_lengths.reshape(N_EDGES, 1)

    edge_feat = _edge_feat(d_col, w1, b1, w2, b2)

    partials = _segment_partials(edge_lengths, edge_index[1], edge_feat)
    node_energy = partials[0] + partials[1]
    return edge_feat, node_energy
